# baseline (speedup 1.0000x reference)
import functools

import jax
import jax.numpy as jnp
from jax import lax
from jax.experimental import pallas as pl
from jax.experimental.pallas import tpu as pltpu
from jax.experimental.pallas import tpu_sc as plsc

T = 2048
D = 2048
E = 8
F = 768
K = 2

BM = 256
NB = 24
NP = NB * BM

NC, NS = 2, 16
NW = NC * NS


_RB = 512


def _router_body(x_ref, gw_ref, i1_ref, i2_ref, w1_ref, w2_ref):
    x = x_ref[...]
    gw = gw_ref[...]
    logits = lax.dot_general(x, gw, (((1,), (1,)), ((), ())),
                             preferred_element_type=jnp.float32)
    iota = lax.broadcasted_iota(jnp.int32, logits.shape, 1)
    m1 = jnp.max(logits, axis=1, keepdims=True)
    i1 = jnp.min(jnp.where(logits == m1, iota, E), axis=1)
    masked = jnp.where(iota == i1[:, None], -jnp.inf, logits)
    m2 = jnp.max(masked, axis=1, keepdims=True)
    i2 = jnp.min(jnp.where(masked == m2, iota, E), axis=1)
    w1 = jax.nn.sigmoid(m1[:, 0] - m2[:, 0])
    i1_ref[...] = i1.astype(jnp.int32)
    i2_ref[...] = i2.astype(jnp.int32)
    w1_ref[...] = w1
    w2_ref[...] = 1.0 - w1


def _router(x, gate_weight):
    return pl.pallas_call(
        _router_body,
        grid=(T // _RB,),
        in_specs=[
            pl.BlockSpec((_RB, D), lambda b: (b, 0)),
            pl.BlockSpec((E, D), lambda b: (0, 0)),
        ],
        out_specs=[
            pl.BlockSpec((_RB,), lambda b: (b,)),
            pl.BlockSpec((_RB,), lambda b: (b,)),
            pl.BlockSpec((_RB,), lambda b: (b,)),
            pl.BlockSpec((_RB,), lambda b: (b,)),
        ],
        out_shape=[
            jax.ShapeDtypeStruct((T,), jnp.int32),
            jax.ShapeDtypeStruct((T,), jnp.int32),
            jax.ShapeDtypeStruct((T,), jnp.float32),
            jax.ShapeDtypeStruct((T,), jnp.float32),
        ],
    )(x, gate_weight)


def _dispatch_metadata(i1, i2, w1, w2):
    flat_e = jnp.stack([i1, i2], axis=1).reshape(-1)
    flat_w = jnp.stack([w1, w2], axis=1).reshape(-1)
    onehot = (flat_e[:, None] == jnp.arange(E)[None, :]).astype(jnp.int32)
    counts = jnp.sum(onehot, axis=0)
    rank = jnp.take_along_axis(jnp.cumsum(onehot, axis=0) - 1,
                               flat_e[:, None], axis=1)[:, 0]
    padded = ((counts + BM - 1) // BM) * BM
    pad_off = jnp.concatenate([jnp.zeros(1, jnp.int32),
                               jnp.cumsum(padded)[:-1]]).astype(jnp.int32)
    dest = pad_off[flat_e] + rank

    tok_padded = jnp.zeros((NP,), jnp.int32).at[dest].set(
        jnp.arange(T * K, dtype=jnp.int32) // K)
    ws_padded = jnp.zeros((NP,), jnp.float32).at[dest].set(flat_w)

    total_padded = jnp.sum(padded)
    nb = total_padded // BM
    b_ids = jnp.arange(NB, dtype=jnp.int32)
    pad_end = pad_off + padded
    be_raw = jnp.sum((b_ids[:, None] * BM >= pad_end[None, :]).astype(jnp.int32),
                     axis=1)
    active = b_ids < nb
    e_last = jnp.take(be_raw, nb - 1)
    block_expert = jnp.where(active, be_raw, e_last).astype(jnp.int32)
    xbi = jnp.where(active, b_ids, nb - 1).astype(jnp.int32)
    act = active.astype(jnp.int32)

    pos = dest.reshape(T, K)
    return tok_padded, ws_padded, block_expert, xbi, act, pos[:, 0], pos[:, 1]


_RPW = NP // NW
_GCH = 48


def _gather_body(x_hbm, tok_hbm, xs_hbm, idx_v, rows_v, sem):
    wid = lax.axis_index("s") * NC + lax.axis_index("c")
    base = wid * _RPW
    pltpu.sync_copy(tok_hbm.at[pl.ds(base, _RPW)], idx_v)

    def chunk(c, _):
        pltpu.async_copy(x_hbm.at[idx_v.at[pl.ds(c * _GCH, _GCH)]],
                         rows_v, sem).wait()
        pltpu.sync_copy(rows_v, xs_hbm.at[pl.ds(base + c * _GCH, _GCH)])
        return 0

    lax.fori_loop(0, _RPW // _GCH, chunk, 0)


def _dispatch_gather(x, tok_padded):
    mesh = plsc.VectorSubcoreMesh(core_axis_name="c", subcore_axis_name="s")
    run = pl.kernel(
        _gather_body,
        out_type=jax.ShapeDtypeStruct((NP, D), jnp.float32),
        mesh=mesh,
        scratch_types=[
            pltpu.VMEM((_RPW,), jnp.int32),
            pltpu.VMEM((_GCH, D), jnp.float32),
            pltpu.SemaphoreType.DMA,
        ],
    )
    return run(x, tok_padded)


def _ffn_body(be_ref, xbi_ref, act_ref, xs_ref, ws_ref,
              wg_ref, wu_ref, wd_ref, ys_ref):
    b = pl.program_id(0)

    @pl.when(act_ref[b] == 1)
    def _():
        xb = xs_ref[...].astype(jnp.bfloat16)
        wg = wg_ref[0].astype(jnp.bfloat16)
        wu = wu_ref[0].astype(jnp.bfloat16)
        wd = wd_ref[0].astype(jnp.bfloat16)
        g = jnp.dot(xb, wg, preferred_element_type=jnp.float32)
        u = jnp.dot(xb, wu, preferred_element_type=jnp.float32)
        h = (g * jax.nn.sigmoid(g)) * u
        y = jnp.dot(h.astype(jnp.bfloat16), wd,
                    preferred_element_type=jnp.float32)
        ys_ref[...] = y * ws_ref[...]


def _grouped_ffn(xs, ws, w_gate, w_up, w_down, be, xbi, act):
    grid_spec = pltpu.PrefetchScalarGridSpec(
        num_scalar_prefetch=3,
        grid=(NB,),
        in_specs=[
            pl.BlockSpec((BM, D), lambda b, be, xbi, act: (xbi[b], 0)),
            pl.BlockSpec((BM, 1), lambda b, be, xbi, act: (xbi[b], 0)),
            pl.BlockSpec((1, D, F), lambda b, be, xbi, act: (be[b], 0, 0)),
            pl.BlockSpec((1, D, F), lambda b, be, xbi, act: (be[b], 0, 0)),
            pl.BlockSpec((1, F, D), lambda b, be, xbi, act: (be[b], 0, 0)),
        ],
        out_specs=pl.BlockSpec((BM, D), lambda b, be, xbi, act: (xbi[b], 0)),
    )
    return pl.pallas_call(
        _ffn_body,
        grid_spec=grid_spec,
        out_shape=jax.ShapeDtypeStruct((NP, D), jnp.float32),
        compiler_params=pltpu.CompilerParams(
            dimension_semantics=("arbitrary",)),
    )(be, xbi, act, xs, ws.reshape(NP, 1), w_gate, w_up, w_down)


_TPW = T // NW
_CCH = 16


def _combine_body(ys_hbm, p1_hbm, p2_hbm, out_hbm,
                  i1_v, i2_v, r1_v, r2_v, sem1, sem2):
    wid = lax.axis_index("s") * NC + lax.axis_index("c")
    base = wid * _TPW
    pltpu.sync_copy(p1_hbm.at[pl.ds(base, _TPW)], i1_v)
    pltpu.sync_copy(p2_hbm.at[pl.ds(base, _TPW)], i2_v)

    def chunk(c, _):
        cp1 = pltpu.async_copy(ys_hbm.at[i1_v.at[pl.ds(c * _CCH, _CCH)]],
                               r1_v, sem1)
        cp2 = pltpu.async_copy(ys_hbm.at[i2_v.at[pl.ds(c * _CCH, _CCH)]],
                               r2_v, sem2)
        cp1.wait()
        cp2.wait()

        def add_row(r, _):
            def add_vec(j, _):
                sl = pl.ds(j * 16, 16)
                r1_v[r, sl] = r1_v[r, sl] + r2_v[r, sl]
                return 0
            lax.fori_loop(0, D // 16, add_vec, 0)
            return 0

        lax.fori_loop(0, _CCH, add_row, 0)
        pltpu.sync_copy(r1_v, out_hbm.at[pl.ds(base + c * _CCH, _CCH)])
        return 0

    lax.fori_loop(0, _TPW // _CCH, chunk, 0)


def _combine(ys, p1, p2):
    mesh = plsc.VectorSubcoreMesh(core_axis_name="c", subcore_axis_name="s")
    run = pl.kernel(
        _combine_body,
        out_type=jax.ShapeDtypeStruct((T, D), jnp.float32),
        mesh=mesh,
        scratch_types=[
            pltpu.VMEM((_TPW,), jnp.int32),
            pltpu.VMEM((_TPW,), jnp.int32),
            pltpu.VMEM((_CCH, D), jnp.float32),
            pltpu.VMEM((_CCH, D), jnp.float32),
            pltpu.SemaphoreType.DMA,
            pltpu.SemaphoreType.DMA,
        ],
    )
    return run(ys, p1, p2)


def kernel(hidden_states, gate_weight, w_gate_proj, w_up_proj, w_down_proj):
    x = hidden_states.reshape(T, D)
    i1, i2, w1, w2 = _router(x, gate_weight)
    tok, ws, be, xbi, act, p1, p2 = _dispatch_metadata(i1, i2, w1, w2)
    xs = _dispatch_gather(x, tok)
    ys = _grouped_ffn(xs, ws, w_gate_proj, w_up_proj, w_down_proj,
                      be, xbi, act)
    out = _combine(ys, p1, p2)
    return out.reshape(hidden_states.shape)


# speedup vs baseline: 1.0322x; 1.0322x over previous
import functools

import jax
import jax.numpy as jnp
from jax import lax
from jax.experimental import pallas as pl
from jax.experimental.pallas import tpu as pltpu
from jax.experimental.pallas import tpu_sc as plsc

T = 2048
D = 2048
E = 8
F = 768
K = 2

BM = 256
NB = 24
NP = NB * BM

NC, NS = 2, 16
NW = NC * NS


_RB = 512


def _router_body(x_ref, gw_ref, i1_ref, i2_ref, w1_ref, w2_ref):
    x = x_ref[...]
    gw = gw_ref[...]
    logits = lax.dot_general(x, gw, (((1,), (1,)), ((), ())),
                             preferred_element_type=jnp.float32)
    iota = lax.broadcasted_iota(jnp.int32, logits.shape, 1)
    m1 = jnp.max(logits, axis=1, keepdims=True)
    i1 = jnp.min(jnp.where(logits == m1, iota, E), axis=1)
    masked = jnp.where(iota == i1[:, None], -jnp.inf, logits)
    m2 = jnp.max(masked, axis=1, keepdims=True)
    i2 = jnp.min(jnp.where(masked == m2, iota, E), axis=1)
    w1 = jax.nn.sigmoid(m1[:, 0] - m2[:, 0])
    i1_ref[...] = i1.astype(jnp.int32)
    i2_ref[...] = i2.astype(jnp.int32)
    w1_ref[...] = w1
    w2_ref[...] = 1.0 - w1


def _router(x, gate_weight):
    return pl.pallas_call(
        _router_body,
        grid=(T // _RB,),
        in_specs=[
            pl.BlockSpec((_RB, D), lambda b: (b, 0)),
            pl.BlockSpec((E, D), lambda b: (0, 0)),
        ],
        out_specs=[
            pl.BlockSpec((_RB,), lambda b: (b,)),
            pl.BlockSpec((_RB,), lambda b: (b,)),
            pl.BlockSpec((_RB,), lambda b: (b,)),
            pl.BlockSpec((_RB,), lambda b: (b,)),
        ],
        out_shape=[
            jax.ShapeDtypeStruct((T,), jnp.int32),
            jax.ShapeDtypeStruct((T,), jnp.int32),
            jax.ShapeDtypeStruct((T,), jnp.float32),
            jax.ShapeDtypeStruct((T,), jnp.float32),
        ],
    )(x, gate_weight)


def _dispatch_metadata(i1, i2, w1, w2):
    flat_e = jnp.stack([i1, i2], axis=1).reshape(-1)
    flat_w = jnp.stack([w1, w2], axis=1).reshape(-1)
    onehot = (flat_e[:, None] == jnp.arange(E)[None, :]).astype(jnp.int32)
    counts = jnp.sum(onehot, axis=0)
    rank = jnp.take_along_axis(jnp.cumsum(onehot, axis=0) - 1,
                               flat_e[:, None], axis=1)[:, 0]
    padded = ((counts + BM - 1) // BM) * BM
    pad_off = jnp.concatenate([jnp.zeros(1, jnp.int32),
                               jnp.cumsum(padded)[:-1]]).astype(jnp.int32)
    dest = pad_off[flat_e] + rank

    tok_padded = jnp.zeros((NP,), jnp.int32).at[dest].set(
        jnp.arange(T * K, dtype=jnp.int32) // K)
    ws_padded = jnp.zeros((NP,), jnp.float32).at[dest].set(flat_w)

    total_padded = jnp.sum(padded)
    nb = total_padded // BM
    b_ids = jnp.arange(NB, dtype=jnp.int32)
    pad_end = pad_off + padded
    be_raw = jnp.sum((b_ids[:, None] * BM >= pad_end[None, :]).astype(jnp.int32),
                     axis=1)
    active = b_ids < nb
    e_last = jnp.take(be_raw, nb - 1)
    block_expert = jnp.where(active, be_raw, e_last).astype(jnp.int32)
    xbi = jnp.where(active, b_ids, nb - 1).astype(jnp.int32)
    act = active.astype(jnp.int32)

    pos = dest.reshape(T, K)
    return tok_padded, ws_padded, block_expert, xbi, act, pos[:, 0], pos[:, 1]


_RPW = NP // NW
_GCH = 16
_GNB = 3
_GNC = _RPW // _GCH


def _gather_body(x_hbm, tok_hbm, xs_hbm, idx_v, b0, b1, b2,
                 g0, g1, g2, w0, w1, w2):
    wid = lax.axis_index("s") * NC + lax.axis_index("c")
    base = wid * _RPW
    pltpu.sync_copy(tok_hbm.at[pl.ds(base, _RPW)], idx_v)
    bufs = (b0, b1, b2)
    gsems = (g0, g1, g2)
    wsems = (w0, w1, w2)

    def fire_gather(c, s):
        return pltpu.async_copy(
            x_hbm.at[idx_v.at[pl.ds(c * _GCH, _GCH)]], bufs[s], gsems[s])

    wb = [None] * _GNB
    gd = [None] * _GNB
    for c in range(min(_GNB, _GNC)):
        gd[c] = fire_gather(c, c)
    for c in range(_GNC):
        s = c % _GNB
        gd[s].wait()
        wb[s] = pltpu.async_copy(
            bufs[s], xs_hbm.at[pl.ds(base + c * _GCH, _GCH)], wsems[s])
        n = c + _GNB
        if n < _GNC:
            wb[s].wait()
            gd[s] = fire_gather(n, s)
    for s in range(min(_GNB, _GNC)):
        if wb[s] is not None:
            wb[s].wait()


def _dispatch_gather(x, tok_padded):
    mesh = plsc.VectorSubcoreMesh(core_axis_name="c", subcore_axis_name="s")
    run = pl.kernel(
        _gather_body,
        out_type=jax.ShapeDtypeStruct((NP, D), jnp.float32),
        mesh=mesh,
        scratch_types=[
            pltpu.VMEM((_RPW,), jnp.int32),
            pltpu.VMEM((_GCH, D), jnp.float32),
            pltpu.VMEM((_GCH, D), jnp.float32),
            pltpu.VMEM((_GCH, D), jnp.float32),
            pltpu.SemaphoreType.DMA,
            pltpu.SemaphoreType.DMA,
            pltpu.SemaphoreType.DMA,
            pltpu.SemaphoreType.DMA,
            pltpu.SemaphoreType.DMA,
            pltpu.SemaphoreType.DMA,
        ],
    )
    return run(x, tok_padded)


def _ffn_body(be_ref, xbi_ref, act_ref, xs_ref, ws_ref,
              wg_ref, wu_ref, wd_ref, ys_ref):
    b = pl.program_id(0)

    @pl.when(act_ref[b] == 1)
    def _():
        xb = xs_ref[...].astype(jnp.bfloat16)
        wg = wg_ref[0].astype(jnp.bfloat16)
        wu = wu_ref[0].astype(jnp.bfloat16)
        wd = wd_ref[0].astype(jnp.bfloat16)
        g = jnp.dot(xb, wg, preferred_element_type=jnp.float32)
        u = jnp.dot(xb, wu, preferred_element_type=jnp.float32)
        h = (g * jax.nn.sigmoid(g)) * u
        y = jnp.dot(h.astype(jnp.bfloat16), wd,
                    preferred_element_type=jnp.float32)
        ys_ref[...] = y * ws_ref[...]


def _grouped_ffn(xs, ws, w_gate, w_up, w_down, be, xbi, act):
    grid_spec = pltpu.PrefetchScalarGridSpec(
        num_scalar_prefetch=3,
        grid=(NB,),
        in_specs=[
            pl.BlockSpec((BM, D), lambda b, be, xbi, act: (xbi[b], 0)),
            pl.BlockSpec((BM, 1), lambda b, be, xbi, act: (xbi[b], 0)),
            pl.BlockSpec((1, D, F), lambda b, be, xbi, act: (be[b], 0, 0)),
            pl.BlockSpec((1, D, F), lambda b, be, xbi, act: (be[b], 0, 0)),
            pl.BlockSpec((1, F, D), lambda b, be, xbi, act: (be[b], 0, 0)),
        ],
        out_specs=pl.BlockSpec((BM, D), lambda b, be, xbi, act: (xbi[b], 0)),
    )
    return pl.pallas_call(
        _ffn_body,
        grid_spec=grid_spec,
        out_shape=jax.ShapeDtypeStruct((NP, D), jnp.float32),
        compiler_params=pltpu.CompilerParams(
            dimension_semantics=("arbitrary",)),
    )(be, xbi, act, xs, ws.reshape(NP, 1), w_gate, w_up, w_down)


_TPW = T // NW
_CCH = 8
_CNC = _TPW // _CCH


def _combine_body(ys_hbm, p1_hbm, p2_hbm, out_hbm, i1_v, i2_v,
                  r1a, r2a, r1b, r2b, sg1a, sg2a, sg1b, sg2b, swa, swb):
    wid = lax.axis_index("s") * NC + lax.axis_index("c")
    base = wid * _TPW
    pltpu.sync_copy(p1_hbm.at[pl.ds(base, _TPW)], i1_v)
    pltpu.sync_copy(p2_hbm.at[pl.ds(base, _TPW)], i2_v)
    r1 = (r1a, r1b)
    r2 = (r2a, r2b)
    sg1 = (sg1a, sg1b)
    sg2 = (sg2a, sg2b)
    sw = (swa, swb)

    def fire(c, s):
        sl = pl.ds(c * _CCH, _CCH)
        return (pltpu.async_copy(ys_hbm.at[i1_v.at[sl]], r1[s], sg1[s]),
                pltpu.async_copy(ys_hbm.at[i2_v.at[sl]], r2[s], sg2[s]))

    gd = [None, None]
    wb = [None, None]
    gd[0] = fire(0, 0)
    for c in range(_CNC):
        s = c % 2
        gd[s][0].wait()
        gd[s][1].wait()
        if c + 1 < _CNC:
            if wb[1 - s] is not None:
                wb[1 - s].wait()
            gd[1 - s] = fire(c + 1, 1 - s)

        def add_row(r, _):
            def add_vec(j, _):
                sl = pl.ds(j * 16, 16)
                r1[s][r, sl] = r1[s][r, sl] + r2[s][r, sl]
                return 0
            lax.fori_loop(0, D // 16, add_vec, 0)
            return 0

        lax.fori_loop(0, _CCH, add_row, 0)
        wb[s] = pltpu.async_copy(
            r1[s], out_hbm.at[pl.ds(base + c * _CCH, _CCH)], sw[s])
    for s in range(2):
        if wb[s] is not None:
            wb[s].wait()


def _combine(ys, p1, p2):
    mesh = plsc.VectorSubcoreMesh(core_axis_name="c", subcore_axis_name="s")
    run = pl.kernel(
        _combine_body,
        out_type=jax.ShapeDtypeStruct((T, D), jnp.float32),
        mesh=mesh,
        scratch_types=[
            pltpu.VMEM((_TPW,), jnp.int32),
            pltpu.VMEM((_TPW,), jnp.int32),
            pltpu.VMEM((_CCH, D), jnp.float32),
            pltpu.VMEM((_CCH, D), jnp.float32),
            pltpu.VMEM((_CCH, D), jnp.float32),
            pltpu.VMEM((_CCH, D), jnp.float32),
            pltpu.SemaphoreType.DMA,
            pltpu.SemaphoreType.DMA,
            pltpu.SemaphoreType.DMA,
            pltpu.SemaphoreType.DMA,
            pltpu.SemaphoreType.DMA,
            pltpu.SemaphoreType.DMA,
        ],
    )
    return run(ys, p1, p2)


def kernel(hidden_states, gate_weight, w_gate_proj, w_up_proj, w_down_proj):
    x = hidden_states.reshape(T, D)
    i1, i2, w1, w2 = _router(x, gate_weight)
    tok, ws, be, xbi, act, p1, p2 = _dispatch_metadata(i1, i2, w1, w2)
    xs = _dispatch_gather(x, tok)
    ys = _grouped_ffn(xs, ws, w_gate_proj, w_up_proj, w_down_proj,
                      be, xbi, act)
    out = _combine(ys, p1, p2)
    return out.reshape(hidden_states.shape)


# speedup vs baseline: 1.2351x; 1.1966x over previous
import functools

import jax
import jax.numpy as jnp
from jax import lax
from jax.experimental import pallas as pl
from jax.experimental.pallas import tpu as pltpu
from jax.experimental.pallas import tpu_sc as plsc

T = 2048
D = 2048
E = 8
F = 768
K = 2

BM = 256
NB = 24
NP = NB * BM

NC, NS = 2, 16
NW = NC * NS


_RB = 512


def _router_body(x_ref, gw_ref, i1_ref, i2_ref, w1_ref, w2_ref, xb_ref):
    x = x_ref[...]
    gw = gw_ref[...]
    def _bf16_bits(v):
        u = lax.bitcast_convert_type(v, jnp.int32)
        return (u + 0x7FFF + ((u >> 16) & 1)) >> 16

    blo = _bf16_bits(x[:, :D // 2]) & 0xFFFF
    bhi = _bf16_bits(x[:, D // 2:])
    xb_ref[...] = blo | (bhi << 16)
    logits = lax.dot_general(x, gw, (((1,), (1,)), ((), ())),
                             preferred_element_type=jnp.float32)
    iota = lax.broadcasted_iota(jnp.int32, logits.shape, 1)
    m1 = jnp.max(logits, axis=1, keepdims=True)
    i1 = jnp.min(jnp.where(logits == m1, iota, E), axis=1)
    masked = jnp.where(iota == i1[:, None], -jnp.inf, logits)
    m2 = jnp.max(masked, axis=1, keepdims=True)
    i2 = jnp.min(jnp.where(masked == m2, iota, E), axis=1)
    w1 = jax.nn.sigmoid(m1[:, 0] - m2[:, 0])
    i1_ref[...] = i1.astype(jnp.int32)
    i2_ref[...] = i2.astype(jnp.int32)
    w1_ref[...] = w1
    w2_ref[...] = 1.0 - w1


def _router(x, gate_weight):
    return pl.pallas_call(
        _router_body,
        grid=(T // _RB,),
        in_specs=[
            pl.BlockSpec((_RB, D), lambda b: (b, 0)),
            pl.BlockSpec((E, D), lambda b: (0, 0)),
        ],
        out_specs=[
            pl.BlockSpec((_RB,), lambda b: (b,)),
            pl.BlockSpec((_RB,), lambda b: (b,)),
            pl.BlockSpec((_RB,), lambda b: (b,)),
            pl.BlockSpec((_RB,), lambda b: (b,)),
            pl.BlockSpec((_RB, D // 2), lambda b: (b, 0)),
        ],
        out_shape=[
            jax.ShapeDtypeStruct((T,), jnp.int32),
            jax.ShapeDtypeStruct((T,), jnp.int32),
            jax.ShapeDtypeStruct((T,), jnp.float32),
            jax.ShapeDtypeStruct((T,), jnp.float32),
            jax.ShapeDtypeStruct((T, D // 2), jnp.int32),
        ],
    )(x, gate_weight)


def _dispatch_metadata(i1, i2, w1, w2):
    flat_e = jnp.stack([i1, i2], axis=1).reshape(-1)
    flat_w = jnp.stack([w1, w2], axis=1).reshape(-1)
    onehot = (flat_e[:, None] == jnp.arange(E)[None, :]).astype(jnp.int32)
    counts = jnp.sum(onehot, axis=0)
    rank = jnp.take_along_axis(jnp.cumsum(onehot, axis=0) - 1,
                               flat_e[:, None], axis=1)[:, 0]
    padded = ((counts + BM - 1) // BM) * BM
    pad_off = jnp.concatenate([jnp.zeros(1, jnp.int32),
                               jnp.cumsum(padded)[:-1]]).astype(jnp.int32)
    dest = pad_off[flat_e] + rank

    tok_padded = jnp.zeros((NP,), jnp.int32).at[dest].set(
        jnp.arange(T * K, dtype=jnp.int32) // K)
    ws_padded = jnp.zeros((NP,), jnp.float32).at[dest].set(flat_w)

    total_padded = jnp.sum(padded)
    nb = total_padded // BM
    b_ids = jnp.arange(NB, dtype=jnp.int32)
    pad_end = pad_off + padded
    be_raw = jnp.sum((b_ids[:, None] * BM >= pad_end[None, :]).astype(jnp.int32),
                     axis=1)
    active = b_ids < nb
    e_last = jnp.take(be_raw, nb - 1)
    block_expert = jnp.where(active, be_raw, e_last).astype(jnp.int32)
    xbi = jnp.where(active, b_ids, nb - 1).astype(jnp.int32)
    act = active.astype(jnp.int32)

    pos = dest.reshape(T, K)
    return tok_padded, ws_padded, block_expert, xbi, act, pos[:, 0], pos[:, 1]


_RPW = NP // NW
_GCH = 8
_GNB = 6
_GNC = _RPW // _GCH
_DH = D // 2


def _gather_body(x_hbm, tok_hbm, xs_hbm, idx_v, *scr):
    bufs = scr[:_GNB]
    gsems = scr[_GNB:2 * _GNB]
    wsems = scr[2 * _GNB:]
    wid = lax.axis_index("s") * NC + lax.axis_index("c")
    base = wid * _RPW
    pltpu.sync_copy(tok_hbm.at[pl.ds(base, _RPW)], idx_v)

    def fire_gather(c, s):
        return pltpu.async_copy(
            x_hbm.at[idx_v.at[pl.ds(c * _GCH, _GCH)]], bufs[s], gsems[s])

    wb = [None] * _GNB
    gd = [None] * _GNB
    for c in range(min(_GNB, _GNC)):
        gd[c] = fire_gather(c, c)
    for c in range(_GNC):
        s = c % _GNB
        gd[s].wait()
        wb[s] = pltpu.async_copy(
            bufs[s], xs_hbm.at[pl.ds(base + c * _GCH, _GCH)], wsems[s])
        n = c + _GNB
        if n < _GNC:
            wb[s].wait()
            gd[s] = fire_gather(n, s)
    for s in range(min(_GNB, _GNC)):
        if wb[s] is not None:
            wb[s].wait()


def _dispatch_gather(xb, tok_padded):
    mesh = plsc.VectorSubcoreMesh(core_axis_name="c", subcore_axis_name="s")
    run = pl.kernel(
        _gather_body,
        out_type=jax.ShapeDtypeStruct((NP, _DH), jnp.int32),
        mesh=mesh,
        scratch_types=(
            [pltpu.VMEM((_RPW,), jnp.int32)]
            + [pltpu.VMEM((_GCH, _DH), jnp.int32)] * _GNB
            + [pltpu.SemaphoreType.DMA] * (2 * _GNB)
        ),
    )
    return run(xb, tok_padded)


def _ffn_body(be_ref, xbi_ref, act_ref, xs_ref, ws_ref,
              wg_ref, wu_ref, wd_ref, ys_ref):
    b = pl.program_id(0)

    @pl.when(act_ref[b] == 1)
    def _():
        packed = xs_ref[...]
        xlo = lax.bitcast_convert_type(packed << 16, jnp.float32)
        xhi = lax.bitcast_convert_type(packed & jnp.int32(-65536),
                                       jnp.float32)
        xb = jnp.concatenate([xlo, xhi], axis=1).astype(jnp.bfloat16)
        wg = wg_ref[0].astype(jnp.bfloat16)
        wu = wu_ref[0].astype(jnp.bfloat16)
        wd = wd_ref[0].astype(jnp.bfloat16)
        g = jnp.dot(xb, wg, preferred_element_type=jnp.float32)
        u = jnp.dot(xb, wu, preferred_element_type=jnp.float32)
        h = (g * jax.nn.sigmoid(g)) * u
        y = jnp.dot(h.astype(jnp.bfloat16), wd,
                    preferred_element_type=jnp.float32)
        ys_ref[...] = y * ws_ref[...]


def _grouped_ffn(xs, ws, w_gate, w_up, w_down, be, xbi, act):
    grid_spec = pltpu.PrefetchScalarGridSpec(
        num_scalar_prefetch=3,
        grid=(NB,),
        in_specs=[
            pl.BlockSpec((BM, _DH), lambda b, be, xbi, act: (xbi[b], 0)),
            pl.BlockSpec((BM, 1), lambda b, be, xbi, act: (xbi[b], 0)),
            pl.BlockSpec((1, D, F), lambda b, be, xbi, act: (be[b], 0, 0)),
            pl.BlockSpec((1, D, F), lambda b, be, xbi, act: (be[b], 0, 0)),
            pl.BlockSpec((1, F, D), lambda b, be, xbi, act: (be[b], 0, 0)),
        ],
        out_specs=pl.BlockSpec((BM, D), lambda b, be, xbi, act: (xbi[b], 0)),
    )
    return pl.pallas_call(
        _ffn_body,
        grid_spec=grid_spec,
        out_shape=jax.ShapeDtypeStruct((NP, D), jnp.float32),
        compiler_params=pltpu.CompilerParams(
            dimension_semantics=("arbitrary",)),
    )(be, xbi, act, xs, ws.reshape(NP, 1), w_gate, w_up, w_down)


_TPW = T // NW
_CCH = 8
_CNC = _TPW // _CCH


_CNB = 3


def _combine_body(ys_hbm, p1_hbm, p2_hbm, out_hbm, i1_v, i2_v, *scr):
    r1 = scr[:_CNB]
    r2 = scr[_CNB:2 * _CNB]
    sg1 = scr[2 * _CNB:3 * _CNB]
    sg2 = scr[3 * _CNB:4 * _CNB]
    sw = scr[4 * _CNB:]
    wid = lax.axis_index("s") * NC + lax.axis_index("c")
    base = wid * _TPW
    pltpu.sync_copy(p1_hbm.at[pl.ds(base, _TPW)], i1_v)
    pltpu.sync_copy(p2_hbm.at[pl.ds(base, _TPW)], i2_v)

    def fire(c, s):
        sl = pl.ds(c * _CCH, _CCH)
        return (pltpu.async_copy(ys_hbm.at[i1_v.at[sl]], r1[s], sg1[s]),
                pltpu.async_copy(ys_hbm.at[i2_v.at[sl]], r2[s], sg2[s]))

    gd = [None] * _CNB
    wb = [None] * _CNB
    for c in range(min(_CNB - 1, _CNC)):
        gd[c] = fire(c, c)
    for c in range(_CNC):
        s = c % _CNB
        n = c + _CNB - 1
        if n < _CNC:
            sn = n % _CNB
            if wb[sn] is not None:
                wb[sn].wait()
            gd[sn] = fire(n, sn)
        gd[s][0].wait()
        gd[s][1].wait()

        def add_row(r, _):
            def add_vec(j, _):
                sl = pl.ds(j * 16, 16)
                r1[s][r, sl] = r1[s][r, sl] + r2[s][r, sl]
                return 0
            lax.fori_loop(0, D // 16, add_vec, 0)
            return 0

        lax.fori_loop(0, _CCH, add_row, 0)
        wb[s] = pltpu.async_copy(
            r1[s], out_hbm.at[pl.ds(base + c * _CCH, _CCH)], sw[s])
    for s in range(_CNB):
        if wb[s] is not None:
            wb[s].wait()


def _combine(ys, p1, p2):
    mesh = plsc.VectorSubcoreMesh(core_axis_name="c", subcore_axis_name="s")
    run = pl.kernel(
        _combine_body,
        out_type=jax.ShapeDtypeStruct((T, D), jnp.float32),
        mesh=mesh,
        scratch_types=(
            [pltpu.VMEM((_TPW,), jnp.int32)] * 2
            + [pltpu.VMEM((_CCH, D), jnp.float32)] * (2 * _CNB)
            + [pltpu.SemaphoreType.DMA] * (3 * _CNB)
        ),
    )
    return run(ys, p1, p2)


def kernel(hidden_states, gate_weight, w_gate_proj, w_up_proj, w_down_proj):
    x = hidden_states.reshape(T, D)
    i1, i2, w1, w2, xb = _router(x, gate_weight)
    tok, ws, be, xbi, act, p1, p2 = _dispatch_metadata(i1, i2, w1, w2)
    xs = _dispatch_gather(xb, tok)
    ys = _grouped_ffn(xs, ws, w_gate_proj, w_up_proj, w_down_proj,
                      be, xbi, act)
    out = _combine(ys, p1, p2)
    return out.reshape(hidden_states.shape)


# speedup vs baseline: 2.1474x; 1.7387x over previous
import jax
import jax.numpy as jnp
from jax import lax
from jax.experimental import pallas as pl
from jax.experimental.pallas import tpu as pltpu
from jax.experimental.pallas import tpu_sc as plsc

T = 2048
D = 2048
E = 8
F = 768
K = 2

BM = 128
NB = 40
NP = NB * BM

NC, NS = 2, 16
NW = NC * NS
_DH = D // 2
_TPW = T // NW


_RB = 512
_NBL = T // _RB


def _router_body(x_ref, gw_ref, w1_ref, w2_ref, xb_ref, pe_ref, po_ref,
                 be_ref, xbi_ref, act_ref, cnt_ref):
    b = pl.program_id(0)
    x = x_ref[...]
    gw = gw_ref[...]
    logits = lax.dot_general(x, gw, (((1,), (1,)), ((), ())),
                             preferred_element_type=jnp.float32)
    iota = lax.broadcasted_iota(jnp.int32, logits.shape, 1)
    m1 = jnp.max(logits, axis=1, keepdims=True)
    i1 = jnp.min(jnp.where(logits == m1, iota, E), axis=1)
    oh1 = (iota == i1[:, None]).astype(jnp.float32)
    masked = jnp.where(oh1 > 0, -jnp.inf, logits)
    m2 = jnp.max(masked, axis=1, keepdims=True)
    i2 = jnp.min(jnp.where(masked == m2, iota, E), axis=1)
    oh2 = (iota == i2[:, None]).astype(jnp.float32)

    @pl.when(b < _NBL)
    def _pass1():
        w1 = jax.nn.sigmoid(m1 - m2)
        w1_ref[...] = jnp.broadcast_to(w1, (_RB, 16))
        w2_ref[...] = jnp.broadcast_to(1.0 - w1, (_RB, 16))

        def _bf16_bits(v):
            u = lax.bitcast_convert_type(v, jnp.int32)
            return (u + 0x7FFF + ((u >> 16) & 1)) >> 16

        blo = _bf16_bits(x[:, :D // 2]) & 0xFFFF
        bhi = _bf16_bits(x[:, D // 2:])
        xb_ref[...] = blo | (bhi << 16)
        cnt_ref[pl.ds(b, 1), :] = jnp.sum(oh1 + oh2, axis=0,
                                          keepdims=True)

    @pl.when(b >= _NBL)
    def _pass2():
        bb = b - _NBL
        rows = cnt_ref[...]
        r_iota = lax.broadcasted_iota(jnp.int32, rows.shape, 0)
        c_base = jnp.sum(jnp.where(r_iota < bb, rows, 0.0),
                         axis=0, keepdims=True)
        totals = jnp.sum(rows, axis=0, keepdims=True)
        tot_i = totals.astype(jnp.int32)
        padded = ((tot_i + BM - 1) // BM) * BM
        padded_f = padded.astype(jnp.float32)
        le_i = lax.broadcasted_iota(jnp.int32, (E, E), 0)
        le_j = lax.broadcasted_iota(jnp.int32, (E, E), 1)
        ltri8 = (le_i < le_j).astype(jnp.float32)
        pad_off = lax.dot_general(padded_f, ltri8,
                                  (((1,), (0,)), ((), ())),
                                  preferred_element_type=jnp.float32)
        tt_i = lax.broadcasted_iota(jnp.int32, (_RB, _RB), 0)
        tt_j = lax.broadcasted_iota(jnp.int32, (_RB, _RB), 1)
        strict = (tt_i > tt_j).astype(jnp.float32)
        p_strict = lax.dot_general(strict, oh1 + oh2,
                                   (((1,), (0,)), ((), ())),
                                   preferred_element_type=jnp.float32)
        m = pad_off + c_base + p_strict
        dest1 = jnp.sum(oh1 * m, axis=1).astype(jnp.int32)
        dest2 = jnp.sum(oh2 * m, axis=1).astype(jnp.int32)
        pe_ref[...] = dest1.reshape(_RB // _TPW, _TPW)
        po_ref[...] = dest2.reshape(_RB // _TPW, _TPW)

        @pl.when(b == 2 * _NBL - 1)
        def _tables():
            pad_end = pad_off + padded_f
            nb = (jnp.sum(padded_f) / BM).astype(jnp.int32)
            bi2 = lax.broadcasted_iota(jnp.int32, (NB, E), 0)
            be_raw = jnp.sum((bi2.astype(jnp.float32) * BM >=
                              pad_end).astype(jnp.int32), axis=1)
            b1 = lax.broadcasted_iota(jnp.int32, (NB,), 0)
            active = b1 < nb
            e_last = jnp.sum(jnp.where(b1 == nb - 1, be_raw, 0))
            be_ref[...] = jnp.where(active, be_raw, e_last).astype(jnp.int32)
            xbi_ref[...] = jnp.where(active, b1, nb - 1).astype(jnp.int32)
            act_ref[...] = active.astype(jnp.int32)


def _router_meta(x, gate_weight):
    wpb = _RB // _TPW
    return pl.pallas_call(
        _router_body,
        grid=(2 * _NBL,),
        in_specs=[
            pl.BlockSpec((_RB, D), lambda b: (b % _NBL, 0)),
            pl.BlockSpec((E, D), lambda b: (0, 0)),
        ],
        out_specs=[
            pl.BlockSpec((_RB, 16), lambda b: (jnp.minimum(b, _NBL - 1), 0)),
            pl.BlockSpec((_RB, 16), lambda b: (jnp.minimum(b, _NBL - 1), 0)),
            pl.BlockSpec((_RB, _DH),
                         lambda b: (jnp.minimum(b, _NBL - 1), 0)),
            pl.BlockSpec((wpb, _TPW),
                         lambda b: (jnp.maximum(b - _NBL, 0), 0)),
            pl.BlockSpec((wpb, _TPW),
                         lambda b: (jnp.maximum(b - _NBL, 0), 0)),
            pl.BlockSpec((NB,), lambda b: (0,)),
            pl.BlockSpec((NB,), lambda b: (0,)),
            pl.BlockSpec((NB,), lambda b: (0,)),
        ],
        out_shape=[
            jax.ShapeDtypeStruct((T, 16), jnp.float32),
            jax.ShapeDtypeStruct((T, 16), jnp.float32),
            jax.ShapeDtypeStruct((T, _DH), jnp.int32),
            jax.ShapeDtypeStruct((NW, _TPW), jnp.int32),
            jax.ShapeDtypeStruct((NW, _TPW), jnp.int32),
            jax.ShapeDtypeStruct((NB,), jnp.int32),
            jax.ShapeDtypeStruct((NB,), jnp.int32),
            jax.ShapeDtypeStruct((NB,), jnp.int32),
        ],
        scratch_shapes=[pltpu.VMEM((_NBL, E), jnp.float32)],
        compiler_params=pltpu.CompilerParams(
            dimension_semantics=("arbitrary",)),
    )(x, gate_weight)


def _scatter_body(xb_hbm, pe_hbm, po_hbm, xs_hbm,
                  ie_v, io_v, rows_v, s1, s2):
    wid = lax.axis_index("s") * NC + lax.axis_index("c")
    tb = wid * _TPW
    pltpu.sync_copy(pe_hbm.at[wid], ie_v)
    pltpu.sync_copy(po_hbm.at[wid], io_v)
    pltpu.sync_copy(xb_hbm.at[pl.ds(tb, _TPW)], rows_v)
    c1 = pltpu.async_copy(rows_v, xs_hbm.at[ie_v], s1)
    c2 = pltpu.async_copy(rows_v, xs_hbm.at[io_v], s2)
    c1.wait()
    c2.wait()


def _dispatch_scatter(xb, pe, po):
    mesh = plsc.VectorSubcoreMesh(core_axis_name="c", subcore_axis_name="s")
    run = pl.kernel(
        _scatter_body,
        out_type=jax.ShapeDtypeStruct((NP, _DH), jnp.int32),
        mesh=mesh,
        scratch_types=[
            pltpu.VMEM((_TPW,), jnp.int32),
            pltpu.VMEM((_TPW,), jnp.int32),
            pltpu.VMEM((_TPW, _DH), jnp.int32),
            pltpu.SemaphoreType.DMA,
            pltpu.SemaphoreType.DMA,
        ],
    )
    return run(xb, pe, po)


def _ffn_body(be_ref, xbi_ref, act_ref, xs_ref,
              wg_ref, wu_ref, wd_ref, ys_ref):
    b = pl.program_id(0)

    @pl.when(act_ref[b] == 1)
    def _():
        packed = xs_ref[...]
        xlo = lax.bitcast_convert_type(packed << 16, jnp.float32)
        xhi = lax.bitcast_convert_type(packed & jnp.int32(-65536),
                                       jnp.float32)
        xb = jnp.concatenate([xlo, xhi], axis=1).astype(jnp.bfloat16)
        wg = wg_ref[0].astype(jnp.bfloat16)
        wu = wu_ref[0].astype(jnp.bfloat16)
        wd = wd_ref[0].astype(jnp.bfloat16)
        g = jnp.dot(xb, wg, preferred_element_type=jnp.float32)
        u = jnp.dot(xb, wu, preferred_element_type=jnp.float32)
        h = (g * jax.nn.sigmoid(g)) * u
        y = jnp.dot(h.astype(jnp.bfloat16), wd,
                    preferred_element_type=jnp.float32)

        def _bf16_bits(v):
            u32 = lax.bitcast_convert_type(v, jnp.int32)
            return (u32 + 0x7FFF + ((u32 >> 16) & 1)) >> 16

        blo = _bf16_bits(y[:, :D // 2]) & 0xFFFF
        bhi = _bf16_bits(y[:, D // 2:])
        ys_ref[...] = blo | (bhi << 16)


def _grouped_ffn(xs, w_gate, w_up, w_down, be, xbi, act):
    grid_spec = pltpu.PrefetchScalarGridSpec(
        num_scalar_prefetch=3,
        grid=(NB,),
        in_specs=[
            pl.BlockSpec((BM, _DH), lambda b, be, xbi, act: (xbi[b], 0)),
            pl.BlockSpec((1, D, F), lambda b, be, xbi, act: (be[b], 0, 0)),
            pl.BlockSpec((1, D, F), lambda b, be, xbi, act: (be[b], 0, 0)),
            pl.BlockSpec((1, F, D), lambda b, be, xbi, act: (be[b], 0, 0)),
        ],
        out_specs=pl.BlockSpec((BM, _DH), lambda b, be, xbi, act: (xbi[b], 0)),
    )
    return pl.pallas_call(
        _ffn_body,
        grid_spec=grid_spec,
        out_shape=jax.ShapeDtypeStruct((NP, _DH), jnp.int32),
        compiler_params=pltpu.CompilerParams(
            dimension_semantics=("arbitrary",)),
    )(be, xbi, act, xs, w_gate, w_up, w_down)


_CCH = 8
_CNC = _TPW // _CCH
_CNB = 3


def _combine_body(ys_hbm, pe_hbm, po_hbm, w1_hbm, w2_hbm, out_hbm,
                  i1_v, i2_v, w1_v, w2_v, *scr):
    r1 = scr[:_CNB]
    r2 = scr[_CNB:2 * _CNB]
    ob = scr[2 * _CNB:3 * _CNB]
    sg1 = scr[3 * _CNB:4 * _CNB]
    sg2 = scr[4 * _CNB:5 * _CNB]
    sw = scr[5 * _CNB:]
    wid = lax.axis_index("s") * NC + lax.axis_index("c")
    base = wid * _TPW
    pltpu.sync_copy(pe_hbm.at[wid], i1_v)
    pltpu.sync_copy(po_hbm.at[wid], i2_v)
    pltpu.sync_copy(w1_hbm.at[pl.ds(base, _TPW)], w1_v)
    pltpu.sync_copy(w2_hbm.at[pl.ds(base, _TPW)], w2_v)

    def fire(c, s):
        sl = pl.ds(c * _CCH, _CCH)
        return (pltpu.async_copy(ys_hbm.at[i1_v.at[sl]], r1[s], sg1[s]),
                pltpu.async_copy(ys_hbm.at[i2_v.at[sl]], r2[s], sg2[s]))

    mhi = jnp.full((16,), -65536, jnp.int32)
    gd = [None] * _CNB
    wb = [None] * _CNB
    for c in range(min(_CNB - 1, _CNC)):
        gd[c] = fire(c, c)
    for c in range(_CNC):
        s = c % _CNB
        n = c + _CNB - 1
        if n < _CNC:
            sn = n % _CNB
            if wb[sn] is not None:
                wb[sn].wait()
            gd[sn] = fire(n, sn)
        gd[s][0].wait()
        gd[s][1].wait()

        def add_row(r, _):
            v1 = w1_v[c * _CCH + r, pl.ds(0, 16)]
            v2 = w2_v[c * _CCH + r, pl.ds(0, 16)]

            def add_vec(j, _):
                sl = pl.ds(j * 16, 16)
                p1v = r1[s][r, sl]
                p2v = r2[s][r, sl]
                bc = lax.bitcast_convert_type
                lo = (bc(p1v << 16, jnp.float32) * v1
                      + bc(p2v << 16, jnp.float32) * v2)
                hi = (bc(p1v & mhi, jnp.float32) * v1
                      + bc(p2v & mhi, jnp.float32) * v2)
                ob[s][r, sl] = lo
                ob[s][r, pl.ds(D // 2 + j * 16, 16)] = hi
                return 0
            lax.fori_loop(0, _DH // 16, add_vec, 0)
            return 0

        lax.fori_loop(0, _CCH, add_row, 0)
        wb[s] = pltpu.async_copy(
            ob[s], out_hbm.at[pl.ds(base + c * _CCH, _CCH)], sw[s])
    for s in range(_CNB):
        if wb[s] is not None:
            wb[s].wait()


def _combine(ys, pe, po, w1, w2):
    mesh = plsc.VectorSubcoreMesh(core_axis_name="c", subcore_axis_name="s")
    run = pl.kernel(
        _combine_body,
        out_type=jax.ShapeDtypeStruct((T, D), jnp.float32),
        mesh=mesh,
        scratch_types=(
            [pltpu.VMEM((_TPW,), jnp.int32)] * 2
            + [pltpu.VMEM((_TPW, 16), jnp.float32)] * 2
            + [pltpu.VMEM((_CCH, _DH), jnp.int32)] * (2 * _CNB)
            + [pltpu.VMEM((_CCH, D), jnp.float32)] * _CNB
            + [pltpu.SemaphoreType.DMA] * (3 * _CNB)
        ),
    )
    return run(ys, pe, po, w1, w2)


def kernel(hidden_states, gate_weight, w_gate_proj, w_up_proj, w_down_proj):
    x = hidden_states.reshape(T, D)
    w1, w2, xb, pe, po, be, xbi, act = _router_meta(x, gate_weight)
    xs = _dispatch_scatter(xb, pe, po)
    ys = _grouped_ffn(xs, w_gate_proj, w_up_proj, w_down_proj, be, xbi, act)
    out = _combine(ys, pe, po, w1, w2)
    return out.reshape(hidden_states.shape)
